# baseline (speedup 1.0000x reference)
import jax
import jax.numpy as jnp
from jax import lax
from jax.experimental import pallas as pl
from jax.experimental.pallas import tpu as pltpu
from jax.experimental.pallas import tpu_sc as plsc

_N_DIM = 32
_R = 10.0
_T = 1.0

_INFO = plsc.get_sparse_core_info()
_NC = _INFO.num_cores
_NS = _INFO.num_subcores
_NW = _NC * _NS
_CHUNK = 128
_HALF_CHUNKS = 4

_PHASE = 1 << 18
_TBLK = 4096


def _transpose_body(x0_ref, x1_ref, x2_ref, x3_ref, o_ref):
  eye = jnp.eye(_N_DIM, dtype=jnp.float32)
  parts = [
      lax.dot_general(x_ref[...], eye, (((0,), (0,)), ((), ())),
                      precision=lax.Precision.HIGHEST)
      for x_ref in (x0_ref, x1_ref, x2_ref, x3_ref)
  ]
  o_ref[...] = jnp.concatenate(parts, axis=1)


def _tc_detranspose(table_t):
  grid = _PHASE // _TBLK
  max_blk = pl.cdiv(table_t.shape[1], _TBLK) - 1
  in_specs = [
      pl.BlockSpec(
          (_N_DIM, _TBLK),
          lambda i, q=q: (0, jnp.minimum(q * grid + i, max_blk)))
      for q in range(4)
  ]
  return pl.pallas_call(
      _transpose_body,
      grid=(grid,),
      in_specs=in_specs,
      out_specs=pl.BlockSpec((_TBLK, 4 * _N_DIM), lambda i: (i, 0)),
      out_shape=jax.ShapeDtypeStruct((_PHASE, 4 * _N_DIM), jnp.float32),
  )(table_t, table_t, table_t, table_t)


def _sc_gather(idx_grp, table128, n_idx):
  b_per_w = n_idx // _NW
  n_chunks = b_per_w // _CHUNK
  n_half = n_chunks // _HALF_CHUNKS
  half_rows = _HALF_CHUNKS * _CHUNK
  mesh = plsc.VectorSubcoreMesh(core_axis_name="c", subcore_axis_name="s")

  def body(idx_hbm, table_hbm, out_hbm, idx_v, rows_v, sem):
    wid = lax.axis_index("s") * _NC + lax.axis_index("c")
    base = wid * b_per_w
    pltpu.sync_copy(idx_hbm.at[wid], idx_v)
    for h in range(n_half):
      copies = []
      for k in range(_HALF_CHUNKS):
        copies.append(
            pltpu.async_copy(
                table_hbm.at[idx_v.at[h * _HALF_CHUNKS + k]],
                rows_v.at[pl.ds(k * _CHUNK, _CHUNK)],
                sem,
            ))
      for c in copies:
        c.wait()
      pltpu.sync_copy(rows_v,
                      out_hbm.at[pl.ds(base + h * half_rows, half_rows)])

  return pl.kernel(
      body,
      out_type=jax.ShapeDtypeStruct((n_idx, 128), jnp.float32),
      mesh=mesh,
      scratch_types=[
          pltpu.VMEM((n_chunks, _CHUNK), jnp.int32),
          pltpu.VMEM((half_rows, 128), jnp.float32),
          pltpu.SemaphoreType.DMA,
      ],
  )(idx_grp, table128)


def _loss_body(u_ref, v_ref, pairs_ref, lab_ref, o_ref):
  xu = u_ref[...]
  xv = v_ref[...]
  pr = pairs_ref[...]
  qu = (pr[:, 0:1] >> 18).astype(jnp.float32)
  qv = (pr[:, 1:2] >> 18).astype(jnp.float32)
  u = jnp.zeros((xu.shape[0], _N_DIM), jnp.float32)
  v = jnp.zeros((xu.shape[0], _N_DIM), jnp.float32)
  for q in range(4):
    qf = jnp.float32(q)
    u = u + jnp.where(qu == qf, 1.0, 0.0) * xu[:, q * 32:q * 32 + 32]
    v = v + jnp.where(qv == qf, 1.0, 0.0) * xv[:, q * 32:q * 32 + 32]
  d2 = jnp.sum((u - v) ** 2, axis=1, keepdims=True)
  nu = jnp.sum(u * u, axis=1, keepdims=True)
  nv = jnp.sum(v * v, axis=1, keepdims=True)
  ret = 1.0 + 2.0 * d2 / ((1.0 - nu) * (1.0 - nv))
  dist = jnp.log(ret + jnp.sqrt(ret * ret - 1.0))
  z = (dist - _R) / _T
  labf = lab_ref[...].astype(jnp.float32)
  loss = jnp.where(labf == 1.0,
                   jnp.log(jnp.exp(z) + 1.0),
                   jnp.log(1.0 + jnp.exp(-z)))
  o_ref[...] = loss


def kernel(pairs, labels, table):
  batch = pairs.shape[0]
  n_idx = 2 * batch
  table128 = _tc_detranspose(table.T)
  flat_idx = jnp.concatenate([pairs[:, 0], pairs[:, 1]])
  idx_grp = (flat_idx & (_PHASE - 1)).reshape(_NW, n_idx // _NW // _CHUNK,
                                              _CHUNK)
  rows = _sc_gather(idx_grp, table128, n_idx)
  lab2 = labels.reshape(batch, 1)
  blk = 1024
  grid = batch // blk
  out = pl.pallas_call(
      _loss_body,
      grid=(grid,),
      in_specs=[
          pl.BlockSpec((blk, 128), lambda i: (i, 0)),
          pl.BlockSpec((blk, 128), lambda i: (i + grid, 0)),
          pl.BlockSpec((blk, 2), lambda i: (i, 0)),
          pl.BlockSpec((blk, 1), lambda i: (i, 0)),
      ],
      out_specs=pl.BlockSpec((blk, 1), lambda i: (i, 0)),
      out_shape=jax.ShapeDtypeStruct((batch, 1), jnp.float32),
  )(rows, rows, pairs, lab2)
  return out.reshape(batch)


# speedup vs baseline: 3.3367x; 3.3367x over previous
import jax
import jax.numpy as jnp
from jax import lax
from jax.experimental import pallas as pl
from jax.experimental.pallas import tpu as pltpu
from jax.experimental.pallas import tpu_sc as plsc

_N_DIM = 32
_R = 10.0
_T = 1.0

_INFO = plsc.get_sparse_core_info()
_NC = _INFO.num_cores
_NS = _INFO.num_subcores
_NW = _NC * _NS
_CHUNK = 128
_HALF_CHUNKS = 4

_PHASE = 1 << 18
_TBLK = 4096


def _transpose_body(x0_ref, x1_ref, x2_ref, x3_ref, o_ref):
  stacked = jnp.concatenate(
      [x_ref[...] for x_ref in (x0_ref, x1_ref, x2_ref, x3_ref)], axis=0)
  o_ref[...] = jnp.swapaxes(stacked, 0, 1)


def _tc_detranspose(table_t):
  grid = _PHASE // _TBLK
  max_blk = pl.cdiv(table_t.shape[1], _TBLK) - 1
  in_specs = [
      pl.BlockSpec(
          (_N_DIM, _TBLK),
          lambda i, q=q: (0, jnp.minimum(q * grid + i, max_blk)))
      for q in range(4)
  ]
  return pl.pallas_call(
      _transpose_body,
      grid=(grid,),
      in_specs=in_specs,
      out_specs=pl.BlockSpec((_TBLK, 4 * _N_DIM), lambda i: (i, 0)),
      out_shape=jax.ShapeDtypeStruct((_PHASE, 4 * _N_DIM), jnp.float32),
  )(table_t, table_t, table_t, table_t)


def _sc_gather(idx_grp, table128, n_idx):
  b_per_w = n_idx // _NW
  n_chunks = b_per_w // _CHUNK
  n_half = n_chunks // _HALF_CHUNKS
  half_rows = _HALF_CHUNKS * _CHUNK
  mesh = plsc.VectorSubcoreMesh(core_axis_name="c", subcore_axis_name="s")

  def body(idx_hbm, table_hbm, out_hbm, idx_v, rows_v, sem):
    wid = lax.axis_index("s") * _NC + lax.axis_index("c")
    base = wid * b_per_w
    pltpu.sync_copy(idx_hbm.at[wid], idx_v)
    for h in range(n_half):
      copies = []
      for k in range(_HALF_CHUNKS):
        copies.append(
            pltpu.async_copy(
                table_hbm.at[idx_v.at[h * _HALF_CHUNKS + k]],
                rows_v.at[pl.ds(k * _CHUNK, _CHUNK)],
                sem,
            ))
      for c in copies:
        c.wait()
      pltpu.sync_copy(rows_v,
                      out_hbm.at[pl.ds(base + h * half_rows, half_rows)])

  return pl.kernel(
      body,
      out_type=jax.ShapeDtypeStruct((n_idx, 128), jnp.float32),
      mesh=mesh,
      scratch_types=[
          pltpu.VMEM((n_chunks, _CHUNK), jnp.int32),
          pltpu.VMEM((half_rows, 128), jnp.float32),
          pltpu.SemaphoreType.DMA,
      ],
  )(idx_grp, table128)


def _loss_body(u_ref, v_ref, pairs_ref, lab_ref, o_ref):
  xu = u_ref[...]
  xv = v_ref[...]
  pr = pairs_ref[...]
  qu = (pr[:, 0:1] >> 18).astype(jnp.float32)
  qv = (pr[:, 1:2] >> 18).astype(jnp.float32)
  u = jnp.zeros((xu.shape[0], _N_DIM), jnp.float32)
  v = jnp.zeros((xu.shape[0], _N_DIM), jnp.float32)
  for q in range(4):
    qf = jnp.float32(q)
    u = u + jnp.where(qu == qf, 1.0, 0.0) * xu[:, q * 32:q * 32 + 32]
    v = v + jnp.where(qv == qf, 1.0, 0.0) * xv[:, q * 32:q * 32 + 32]
  d2 = jnp.sum((u - v) ** 2, axis=1, keepdims=True)
  nu = jnp.sum(u * u, axis=1, keepdims=True)
  nv = jnp.sum(v * v, axis=1, keepdims=True)
  ret = 1.0 + 2.0 * d2 / ((1.0 - nu) * (1.0 - nv))
  dist = jnp.log(ret + jnp.sqrt(ret * ret - 1.0))
  z = (dist - _R) / _T
  labf = lab_ref[...].astype(jnp.float32)
  loss = jnp.where(labf == 1.0,
                   jnp.log(jnp.exp(z) + 1.0),
                   jnp.log(1.0 + jnp.exp(-z)))
  o_ref[...] = loss


def kernel(pairs, labels, table):
  batch = pairs.shape[0]
  n_idx = 2 * batch
  table128 = _tc_detranspose(table.T)
  flat_idx = jnp.concatenate([pairs[:, 0], pairs[:, 1]])
  idx_grp = (flat_idx & (_PHASE - 1)).reshape(_NW, n_idx // _NW // _CHUNK,
                                              _CHUNK)
  rows = _sc_gather(idx_grp, table128, n_idx)
  lab2 = labels.reshape(batch, 1)
  blk = 1024
  grid = batch // blk
  out = pl.pallas_call(
      _loss_body,
      grid=(grid,),
      in_specs=[
          pl.BlockSpec((blk, 128), lambda i: (i, 0)),
          pl.BlockSpec((blk, 128), lambda i: (i + grid, 0)),
          pl.BlockSpec((blk, 2), lambda i: (i, 0)),
          pl.BlockSpec((blk, 1), lambda i: (i, 0)),
      ],
      out_specs=pl.BlockSpec((blk, 1), lambda i: (i, 0)),
      out_shape=jax.ShapeDtypeStruct((batch, 1), jnp.float32),
  )(rows, rows, pairs, lab2)
  return out.reshape(batch)


# speedup vs baseline: 3.8453x; 1.1524x over previous
import jax
import jax.numpy as jnp
from jax import lax
from jax.experimental import pallas as pl
from jax.experimental.pallas import tpu as pltpu
from jax.experimental.pallas import tpu_sc as plsc

_N_DIM = 32
_R = 10.0
_T = 1.0

_INFO = plsc.get_sparse_core_info()
_NC = _INFO.num_cores
_NS = _INFO.num_subcores
_NW = _NC * _NS
_CHUNK = 128
_HALF_CHUNKS = 4

_PHASE = 1 << 18
_TBLK = 8192


def _transpose_body(x0_ref, x1_ref, x2_ref, x3_ref, o_ref):
  stacked = jnp.concatenate(
      [x_ref[...] for x_ref in (x0_ref, x1_ref, x2_ref, x3_ref)], axis=0)
  o_ref[...] = jnp.swapaxes(stacked, 0, 1)


def _tc_detranspose(table_t):
  grid = _PHASE // _TBLK
  max_blk = pl.cdiv(table_t.shape[1], _TBLK) - 1
  in_specs = [
      pl.BlockSpec(
          (_N_DIM, _TBLK),
          lambda i, q=q: (0, jnp.minimum(q * grid + i, max_blk)))
      for q in range(4)
  ]
  return pl.pallas_call(
      _transpose_body,
      grid=(grid,),
      in_specs=in_specs,
      out_specs=pl.BlockSpec((_TBLK, 4 * _N_DIM), lambda i: (i, 0)),
      out_shape=jax.ShapeDtypeStruct((_PHASE, 4 * _N_DIM), jnp.float32),
  )(table_t, table_t, table_t, table_t)


def _sc_gather(idx_grp, table128, n_idx):
  b_per_w = n_idx // _NW
  n_chunks = b_per_w // _CHUNK
  n_half = n_chunks // _HALF_CHUNKS
  half_rows = _HALF_CHUNKS * _CHUNK
  mesh = plsc.VectorSubcoreMesh(core_axis_name="c", subcore_axis_name="s")

  def body(idx_hbm, table_hbm, out_hbm, idx_v, rows_v, sem):
    wid = lax.axis_index("s") * _NC + lax.axis_index("c")
    base = wid * b_per_w
    pltpu.sync_copy(idx_hbm.at[wid], idx_v)
    for h in range(n_half):
      copies = []
      for k in range(_HALF_CHUNKS):
        copies.append(
            pltpu.async_copy(
                table_hbm.at[idx_v.at[h * _HALF_CHUNKS + k]],
                rows_v.at[pl.ds(k * _CHUNK, _CHUNK)],
                sem,
            ))
      for c in copies:
        c.wait()
      pltpu.sync_copy(rows_v,
                      out_hbm.at[pl.ds(base + h * half_rows, half_rows)])

  return pl.kernel(
      body,
      out_type=jax.ShapeDtypeStruct((n_idx, 128), jnp.float32),
      mesh=mesh,
      scratch_types=[
          pltpu.VMEM((n_chunks, _CHUNK), jnp.int32),
          pltpu.VMEM((half_rows, 128), jnp.float32),
          pltpu.SemaphoreType.DMA,
      ],
  )(idx_grp, table128)


def _loss_body(u_ref, v_ref, pairs_ref, lab_ref, o_ref):
  xu = u_ref[...]
  xv = v_ref[...]
  pr = pairs_ref[...]
  qu = pr[:, 0:1] >> 18
  qv = pr[:, 1:2] >> 18
  iot = lax.broadcasted_iota(jnp.int32, (xu.shape[0], _N_DIM), 1)
  u = jnp.take_along_axis(xu, qu * _N_DIM + iot, axis=1)
  v = jnp.take_along_axis(xv, qv * _N_DIM + iot, axis=1)
  d2 = jnp.sum((u - v) ** 2, axis=1, keepdims=True)
  nu = jnp.sum(u * u, axis=1, keepdims=True)
  nv = jnp.sum(v * v, axis=1, keepdims=True)
  ret = 1.0 + 2.0 * d2 / ((1.0 - nu) * (1.0 - nv))
  dist = jnp.log(ret + jnp.sqrt(ret * ret - 1.0))
  z = (dist - _R) / _T
  labf = lab_ref[...].astype(jnp.float32)
  loss = jnp.where(labf == 1.0,
                   jnp.log(jnp.exp(z) + 1.0),
                   jnp.log(1.0 + jnp.exp(-z)))
  o_ref[...] = loss


def kernel(pairs, labels, table):
  batch = pairs.shape[0]
  n_idx = 2 * batch
  table128 = _tc_detranspose(table.T)
  flat_idx = jnp.concatenate([pairs[:, 0], pairs[:, 1]])
  idx_grp = (flat_idx & (_PHASE - 1)).reshape(_NW, n_idx // _NW // _CHUNK,
                                              _CHUNK)
  rows = _sc_gather(idx_grp, table128, n_idx)
  lab2 = labels.reshape(batch, 1)
  blk = 1024
  grid = batch // blk
  out = pl.pallas_call(
      _loss_body,
      grid=(grid,),
      in_specs=[
          pl.BlockSpec((blk, 128), lambda i: (i, 0)),
          pl.BlockSpec((blk, 128), lambda i: (i + grid, 0)),
          pl.BlockSpec((blk, 2), lambda i: (i, 0)),
          pl.BlockSpec((blk, 1), lambda i: (i, 0)),
      ],
      out_specs=pl.BlockSpec((blk, 1), lambda i: (i, 0)),
      out_shape=jax.ShapeDtypeStruct((batch, 1), jnp.float32),
  )(rows, rows, pairs, lab2)
  return out.reshape(batch)


# speedup vs baseline: 3.9154x; 1.0182x over previous
import jax
import jax.numpy as jnp
from jax import lax
from jax.experimental import pallas as pl
from jax.experimental.pallas import tpu as pltpu
from jax.experimental.pallas import tpu_sc as plsc

_N_DIM = 32
_R = 10.0
_T = 1.0

_INFO = plsc.get_sparse_core_info()
_NC = _INFO.num_cores
_NS = _INFO.num_subcores
_NW = _NC * _NS
_CHUNK = 128
_HALF_CHUNKS = 4

_PHASE = 1 << 18
_TBLK = 16384


def _transpose_body(x0_ref, x1_ref, x2_ref, x3_ref, o_ref):
  stacked = jnp.concatenate(
      [x_ref[...] for x_ref in (x0_ref, x1_ref, x2_ref, x3_ref)], axis=0)
  o_ref[...] = jnp.swapaxes(stacked, 0, 1)


def _tc_detranspose(table_t):
  grid = _PHASE // _TBLK
  max_blk = pl.cdiv(table_t.shape[1], _TBLK) - 1
  in_specs = [
      pl.BlockSpec(
          (_N_DIM, _TBLK),
          lambda i, q=q: (0, jnp.minimum(q * grid + i, max_blk)))
      for q in range(4)
  ]
  return pl.pallas_call(
      _transpose_body,
      grid=(grid,),
      in_specs=in_specs,
      out_specs=pl.BlockSpec((_TBLK, 4 * _N_DIM), lambda i: (i, 0)),
      out_shape=jax.ShapeDtypeStruct((_PHASE, 4 * _N_DIM), jnp.float32),
  )(table_t, table_t, table_t, table_t)


def _sc_gather(idx_grp, table128, n_idx):
  b_per_w = n_idx // _NW
  n_chunks = b_per_w // _CHUNK
  n_half = n_chunks // _HALF_CHUNKS
  half_rows = _HALF_CHUNKS * _CHUNK
  mesh = plsc.VectorSubcoreMesh(core_axis_name="c", subcore_axis_name="s")

  def body(idx_hbm, table_hbm, out_hbm, idx_v, rows_v, sem):
    wid = lax.axis_index("s") * _NC + lax.axis_index("c")
    base = wid * b_per_w
    pltpu.sync_copy(idx_hbm.at[wid], idx_v)
    for h in range(n_half):
      copies = []
      for k in range(_HALF_CHUNKS):
        copies.append(
            pltpu.async_copy(
                table_hbm.at[idx_v.at[h * _HALF_CHUNKS + k]],
                rows_v.at[pl.ds(k * _CHUNK, _CHUNK)],
                sem,
            ))
      for c in copies:
        c.wait()
      pltpu.sync_copy(rows_v,
                      out_hbm.at[pl.ds(base + h * half_rows, half_rows)])

  return pl.kernel(
      body,
      out_type=jax.ShapeDtypeStruct((n_idx, 128), jnp.float32),
      mesh=mesh,
      scratch_types=[
          pltpu.VMEM((n_chunks, _CHUNK), jnp.int32),
          pltpu.VMEM((half_rows, 128), jnp.float32),
          pltpu.SemaphoreType.DMA,
      ],
  )(idx_grp, table128)


def _loss_body(u_ref, v_ref, pairs_ref, lab_ref, o_ref):
  xu = u_ref[...]
  xv = v_ref[...]
  pr = pairs_ref[...]
  qu = pr[:, 0:1] >> 18
  qv = pr[:, 1:2] >> 18
  iot = lax.broadcasted_iota(jnp.int32, (xu.shape[0], _N_DIM), 1)
  u = jnp.take_along_axis(xu, qu * _N_DIM + iot, axis=1)
  v = jnp.take_along_axis(xv, qv * _N_DIM + iot, axis=1)
  d2 = jnp.sum((u - v) ** 2, axis=1, keepdims=True)
  nu = jnp.sum(u * u, axis=1, keepdims=True)
  nv = jnp.sum(v * v, axis=1, keepdims=True)
  ret = 1.0 + 2.0 * d2 / ((1.0 - nu) * (1.0 - nv))
  dist = jnp.log(ret + jnp.sqrt(ret * ret - 1.0))
  z = (dist - _R) / _T
  labf = lab_ref[...].astype(jnp.float32)
  loss = jnp.where(labf == 1.0,
                   jnp.log(jnp.exp(z) + 1.0),
                   jnp.log(1.0 + jnp.exp(-z)))
  o_ref[...] = loss


def kernel(pairs, labels, table):
  batch = pairs.shape[0]
  n_idx = 2 * batch
  table128 = _tc_detranspose(table.T)
  flat_idx = jnp.concatenate([pairs[:, 0], pairs[:, 1]])
  idx_grp = (flat_idx & (_PHASE - 1)).reshape(_NW, n_idx // _NW // _CHUNK,
                                              _CHUNK)
  rows = _sc_gather(idx_grp, table128, n_idx)
  lab2 = labels.reshape(batch, 1)
  blk = 2048
  grid = batch // blk
  out = pl.pallas_call(
      _loss_body,
      grid=(grid,),
      in_specs=[
          pl.BlockSpec((blk, 128), lambda i: (i, 0)),
          pl.BlockSpec((blk, 128), lambda i: (i + grid, 0)),
          pl.BlockSpec((blk, 2), lambda i: (i, 0)),
          pl.BlockSpec((blk, 1), lambda i: (i, 0)),
      ],
      out_specs=pl.BlockSpec((blk, 1), lambda i: (i, 0)),
      out_shape=jax.ShapeDtypeStruct((batch, 1), jnp.float32),
  )(rows, rows, pairs, lab2)
  return out.reshape(batch)
